# Initial kernel scaffold; baseline (speedup 1.0000x reference)
import jax
import jax.numpy as jnp
from jax.experimental import pallas as pl


def kernel(x, target):
    raise NotImplementedError("write your pallas kernel here")


# speedup vs baseline: 4.3763x; 4.3763x over previous
import jax
import jax.numpy as jnp
from jax.experimental import pallas as pl
from jax.experimental.pallas import tpu as pltpu

_MARGIN = 1.0
_BR = 256


def _triplet_tile(xr_ref, xa_ref, sq_ref, tr_ref, tc_ref, out_ref):
    i = pl.program_id(0)
    xr = xr_ref[...]
    xa = xa_ref[...]
    sq_a = sq_ref[...]
    tgt_r = tr_ref[...]
    tgt_c = tc_ref[...]
    br, n = xr.shape[0], xa.shape[0]

    dot = jax.lax.dot_general(
        xr, xa, (((1,), (1,)), ((), ())), preferred_element_type=jnp.float32
    )
    sq_r = jnp.sum(xr * xr, axis=1, keepdims=True)
    d2 = jnp.maximum(sq_r + sq_a - 2.0 * dot, 0.0)

    same = tgt_r == tgt_c
    col = jax.lax.broadcasted_iota(jnp.int32, (br, n), 1)
    row = jax.lax.broadcasted_iota(jnp.int32, (br, n), 0) + i * br
    not_self = col != row

    posv = jnp.where(same & not_self, d2, -1e9)
    negv = jnp.where(same, 1e9, d2)
    mp = jnp.max(posv, axis=1, keepdims=True)
    mn = jnp.min(negv, axis=1, keepdims=True)

    d2_0 = d2[:, 0:1]
    ap2 = jnp.where(mp < -1e8, d2_0, mp)
    an2 = jnp.where(mn > 1e8, d2_0, mn)

    ap = jnp.sqrt(ap2 + 1e-12)
    an = jnp.sqrt(an2 + 1e-12)
    loss = jnp.maximum(ap - an + _MARGIN, 0.0)
    psum = jnp.sum(loss, axis=(0, 1), keepdims=True)

    @pl.when(i == 0)
    def _():
        out_ref[...] = jnp.zeros((1, 1), jnp.float32)

    out_ref[...] += psum


def kernel(x, target):
    n, d = x.shape
    target = target.astype(jnp.int32)
    sq = jnp.sum(x * x, axis=1).reshape(1, n)
    tgt_col = target.reshape(n, 1)
    tgt_row = target.reshape(1, n)
    grid = (n // _BR,)

    total = pl.pallas_call(
        _triplet_tile,
        grid=grid,
        in_specs=[
            pl.BlockSpec((_BR, d), lambda i: (i, 0)),
            pl.BlockSpec((n, d), lambda i: (0, 0)),
            pl.BlockSpec((1, n), lambda i: (0, 0)),
            pl.BlockSpec((_BR, 1), lambda i: (i, 0)),
            pl.BlockSpec((1, n), lambda i: (0, 0)),
        ],
        out_specs=pl.BlockSpec((1, 1), lambda i: (0, 0)),
        out_shape=jax.ShapeDtypeStruct((1, 1), jnp.float32),
        compiler_params=pltpu.CompilerParams(
            dimension_semantics=("arbitrary",),
        ),
    )(x, x, sq, tgt_col, tgt_row)

    loss_mean = total[0, 0] / n
    return (loss_mean, jnp.asarray(n, dtype=jnp.int32))


# speedup vs baseline: 5.9581x; 1.3614x over previous
import jax
import jax.numpy as jnp
from jax.experimental import pallas as pl
from jax.experimental.pallas import tpu as pltpu

_MARGIN = 1.0
_BR = 256


def _triplet_tile(u_ref, v_ref, tr_ref, tc_ref, out_ref):
    i = pl.program_id(0)
    u = u_ref[...]
    v = v_ref[...]
    tgt_r = tr_ref[...]
    tgt_c = tc_ref[...]
    br, n = u.shape[0], v.shape[0]

    d2 = jax.lax.dot_general(
        u, v, (((1,), (1,)), ((), ())), preferred_element_type=jnp.float32
    )

    same = tgt_r == tgt_c
    col = jax.lax.broadcasted_iota(jnp.int32, (br, n), 1)
    row = jax.lax.broadcasted_iota(jnp.int32, (br, n), 0) + i * br
    not_self = col != row

    posv = jnp.where(same & not_self, d2, -1e9)
    negv = jnp.where(same, 1e9, d2)
    mp = jnp.max(posv, axis=1, keepdims=True)
    mn = jnp.min(negv, axis=1, keepdims=True)

    d2_0 = d2[:, 0:1]
    ap2 = jnp.maximum(jnp.where(mp < -1e8, d2_0, mp), 0.0)
    an2 = jnp.maximum(jnp.where(mn > 1e8, d2_0, mn), 0.0)

    ap = jnp.sqrt(ap2 + 1e-12)
    an = jnp.sqrt(an2 + 1e-12)
    loss = jnp.maximum(ap - an + _MARGIN, 0.0)
    psum = jnp.sum(loss, axis=(0, 1), keepdims=True)

    @pl.when(i == 0)
    def _():
        out_ref[...] = jnp.zeros((1, 1), jnp.float32)

    out_ref[...] += psum


def kernel(x, target):
    n, d = x.shape
    target = target.astype(jnp.int32)
    x_bf = x.astype(jnp.bfloat16)
    sq = jnp.sum(x * x, axis=1)
    sq_hi = sq.astype(jnp.bfloat16)
    sq_lo = (sq - sq_hi.astype(jnp.float32)).astype(jnp.bfloat16)
    one = jnp.ones((n, 1), jnp.bfloat16)
    u = jnp.concatenate(
        [-2.0 * x_bf, sq_hi[:, None], sq_lo[:, None], one, one], axis=1)
    v = jnp.concatenate(
        [x_bf, one, one, sq_hi[:, None], sq_lo[:, None]], axis=1)
    k = d + 4
    tgt_col = target.reshape(n, 1)
    tgt_row = target.reshape(1, n)
    grid = (n // _BR,)

    total = pl.pallas_call(
        _triplet_tile,
        grid=grid,
        in_specs=[
            pl.BlockSpec((_BR, k), lambda i: (i, 0)),
            pl.BlockSpec((n, k), lambda i: (0, 0)),
            pl.BlockSpec((_BR, 1), lambda i: (i, 0)),
            pl.BlockSpec((1, n), lambda i: (0, 0)),
        ],
        out_specs=pl.BlockSpec((1, 1), lambda i: (0, 0)),
        out_shape=jax.ShapeDtypeStruct((1, 1), jnp.float32),
        compiler_params=pltpu.CompilerParams(
            dimension_semantics=("arbitrary",),
        ),
    )(u, v, tgt_col, tgt_row)

    loss_mean = total[0, 0] / n
    return (loss_mean, jnp.asarray(n, dtype=jnp.int32))
